# baseline (speedup 1.0000x reference)
import functools

import jax
import jax.numpy as jnp
from jax import lax
from jax.experimental import pallas as pl
from jax.experimental.pallas import tpu as pltpu
from jax.experimental.pallas import tpu_sc as plsc

N = 262144
C = 100
NC = 2
NS = 16
NW = NC * NS
CPW = N // NW
Q = 256
NG = CPW // Q
NBUF = 2
T = NG // NBUF
VPR = Q // 16


def _body(feat_hbm, fill2_hbm, out_hbm, ins0, ins1, outs0, outs1, fillv,
          lsem0, lsem1, ssem0, ssem1):
    ins = (ins0, ins1)
    outs = (outs0, outs1)
    lsems = (lsem0, lsem1)
    ssems = (ssem0, ssem1)

    wid = lax.axis_index("s") * NC + lax.axis_index("c")
    base = wid * CPW
    pltpu.sync_copy(fill2_hbm, fillv)

    def in_slice(g):
        return feat_hbm.at[:, pl.ds(pl.multiple_of(base + g * Q, 128), Q)]

    def out_slice(g):
        return out_hbm.at[:, pl.ds(pl.multiple_of(base + g * Q, 128), Q)]

    def compute(b):
        def rows(h, carry):
            c = 2 * h
            for dc in range(2):
                f = fillv[c + dc, pl.ds(0, 16)]
                for k in range(VPR):
                    x = ins[b][c + dc, pl.ds(16 * k, 16)]
                    outs[b][c + dc, pl.ds(16 * k, 16)] = (
                        jnp.where(x != x, f, x))
            return carry
        lax.fori_loop(0, C // 2, rows, 0)

    for b in range(NBUF):
        pltpu.make_async_copy(in_slice(b), ins[b], lsems[b]).start()

    for b in range(NBUF):
        g = b
        pltpu.make_async_copy(in_slice(g), ins[b], lsems[b]).wait()
        compute(b)
        pltpu.make_async_copy(outs[b], out_slice(g), ssems[b]).start()
        pltpu.make_async_copy(in_slice(g + NBUF), ins[b], lsems[b]).start()

    def round_body(t, carry):
        for b in range(NBUF):
            g = t * NBUF + b
            pltpu.make_async_copy(in_slice(g), ins[b], lsems[b]).wait()
            pltpu.make_async_copy(outs[b], out_slice(g - NBUF), ssems[b]).wait()
            compute(b)
            pltpu.make_async_copy(outs[b], out_slice(g), ssems[b]).start()
            pltpu.make_async_copy(in_slice(g + NBUF), ins[b], lsems[b]).start()
        return carry

    lax.fori_loop(1, T - 1, round_body, 0)

    for b in range(NBUF):
        g = (T - 1) * NBUF + b
        pltpu.make_async_copy(in_slice(g), ins[b], lsems[b]).wait()
        pltpu.make_async_copy(outs[b], out_slice(g - NBUF), ssems[b]).wait()
        compute(b)
        pltpu.make_async_copy(outs[b], out_slice(g), ssems[b]).start()

    for b in range(NBUF):
        g = (T - 1) * NBUF + b
        pltpu.make_async_copy(outs[b], out_slice(g), ssems[b]).wait()


@jax.jit
def _sc_fill(feat_t, fill2):
    mesh = plsc.VectorSubcoreMesh(core_axis_name="c", subcore_axis_name="s")
    fn = functools.partial(
        pl.kernel,
        mesh=mesh,
        out_type=jax.ShapeDtypeStruct((C, N), jnp.float32),
        scratch_types=[
            pltpu.VMEM((C, Q), jnp.float32),
            pltpu.VMEM((C, Q), jnp.float32),
            pltpu.VMEM((C, Q), jnp.float32),
            pltpu.VMEM((C, Q), jnp.float32),
            pltpu.VMEM((C, 16), jnp.float32),
            pltpu.SemaphoreType.DMA,
            pltpu.SemaphoreType.DMA,
            pltpu.SemaphoreType.DMA,
            pltpu.SemaphoreType.DMA,
        ],
    )(_body)
    return fn(feat_t, fill2)


def kernel(feat, fill_values):
    fv = jnp.where(jnp.isnan(fill_values), 0.0, fill_values)
    fill2 = jnp.tile(fv[:, None], (1, 16))
    return _sc_fill(feat.T, fill2).T


# speedup vs baseline: 1.0067x; 1.0067x over previous
import functools

import jax
import jax.numpy as jnp
from jax import lax
from jax.experimental import pallas as pl
from jax.experimental.pallas import tpu as pltpu
from jax.experimental.pallas import tpu_sc as plsc

N = 262144
C = 100
NC = 2
NS = 16
NW = NC * NS
CPW = N // NW
Q = 256
NG = CPW // Q
NBUF = 2
T = NG // NBUF
VPR = Q // 16


def _body(feat_hbm, fill2_hbm, out_hbm, ins0, ins1, outs0, outs1, fillv,
          lsem0, lsem1, ssem0, ssem1):
    ins = (ins0, ins1)
    outs = (outs0, outs1)
    lsems = (lsem0, lsem1)
    ssems = (ssem0, ssem1)

    wid = lax.axis_index("s") * NC + lax.axis_index("c")
    base = wid * CPW
    pltpu.sync_copy(fill2_hbm, fillv)

    def in_slice(g):
        return feat_hbm.at[:, pl.ds(pl.multiple_of(base + g * Q, 128), Q)]

    def out_slice(g):
        return out_hbm.at[:, pl.ds(pl.multiple_of(base + g * Q, 128), Q)]

    def compute(b):
        def row(c, carry):
            f = fillv[c, pl.ds(0, 16)]
            for k in range(VPR):
                x = ins[b][c, pl.ds(16 * k, 16)]
                outs[b][c, pl.ds(16 * k, 16)] = jnp.where(x != x, f, x)
            return carry
        lax.fori_loop(0, C, row, 0)

    for b in range(NBUF):
        pltpu.make_async_copy(in_slice(b), ins[b], lsems[b]).start()

    for b in range(NBUF):
        g = b
        pltpu.make_async_copy(in_slice(g), ins[b], lsems[b]).wait()
        compute(b)
        pltpu.make_async_copy(outs[b], out_slice(g), ssems[b]).start()
        pltpu.make_async_copy(in_slice(g + NBUF), ins[b], lsems[b]).start()

    def round_body(t, carry):
        for b in range(NBUF):
            g = t * NBUF + b
            pltpu.make_async_copy(in_slice(g), ins[b], lsems[b]).wait()
            pltpu.make_async_copy(outs[b], out_slice(g - NBUF), ssems[b]).wait()
            compute(b)
            pltpu.make_async_copy(outs[b], out_slice(g), ssems[b]).start()
            pltpu.make_async_copy(in_slice(g + NBUF), ins[b], lsems[b]).start()
        return carry

    lax.fori_loop(1, T - 1, round_body, 0)

    for b in range(NBUF):
        g = (T - 1) * NBUF + b
        pltpu.make_async_copy(in_slice(g), ins[b], lsems[b]).wait()
        pltpu.make_async_copy(outs[b], out_slice(g - NBUF), ssems[b]).wait()
        compute(b)
        pltpu.make_async_copy(outs[b], out_slice(g), ssems[b]).start()

    for b in range(NBUF):
        g = (T - 1) * NBUF + b
        pltpu.make_async_copy(outs[b], out_slice(g), ssems[b]).wait()


@jax.jit
def _sc_fill(feat_t, fill2):
    mesh = plsc.VectorSubcoreMesh(core_axis_name="c", subcore_axis_name="s")
    fn = functools.partial(
        pl.kernel,
        mesh=mesh,
        out_type=jax.ShapeDtypeStruct((C, N), jnp.float32),
        scratch_types=[
            pltpu.VMEM((C, Q), jnp.float32),
            pltpu.VMEM((C, Q), jnp.float32),
            pltpu.VMEM((C, Q), jnp.float32),
            pltpu.VMEM((C, Q), jnp.float32),
            pltpu.VMEM((C, 16), jnp.float32),
            pltpu.SemaphoreType.DMA,
            pltpu.SemaphoreType.DMA,
            pltpu.SemaphoreType.DMA,
            pltpu.SemaphoreType.DMA,
        ],
    )(_body)
    return fn(feat_t, fill2)


def kernel(feat, fill_values):
    fv = jnp.where(jnp.isnan(fill_values), 0.0, fill_values)
    fill2 = jnp.tile(fv[:, None], (1, 16))
    return _sc_fill(feat.T, fill2).T
